# baseline (speedup 1.0000x reference)
import jax
import jax.numpy as jnp
from jax import lax
from jax.experimental import pallas as pl
from jax.experimental.pallas import tpu as pltpu
from jax.experimental.pallas import tpu_sc as plsc


def _copy_body(n_rows, chunk, emb_hbm, out_hbm, buf, sem):
    c = lax.axis_index("c")
    s = lax.axis_index("s")
    wid = s * 2 + c
    start = jnp.minimum(wid * chunk, n_rows - chunk)
    pltpu.async_copy(
        emb_hbm.at[pl.ds(start, chunk)], out_hbm.at[pl.ds(start, chunk)], sem
    ).wait()


def kernel(x, emb_table):
    n = x.shape[2]
    d = emb_table.shape[1]
    mesh = plsc.VectorSubcoreMesh(core_axis_name="c", subcore_axis_name="s")
    n_workers = 32
    chunk = (-(-n // n_workers) + 7) // 8 * 8

    def body(emb_hbm, out_hbm, buf, sem):
        _copy_body(n, chunk, emb_hbm, out_hbm, buf, sem)

    out = pl.kernel(
        body,
        out_type=jax.ShapeDtypeStruct((n, d), emb_table.dtype),
        mesh=mesh,
        scratch_types=[
            pltpu.VMEM((chunk, d), emb_table.dtype),
            pltpu.SemaphoreType.DMA,
        ],
    )(emb_table)
    return out[None, None]


# speedup vs baseline: 7.6190x; 7.6190x over previous
import jax
import jax.numpy as jnp
from jax import lax
from jax.experimental import pallas as pl
from jax.experimental.pallas import tpu as pltpu
from jax.experimental.pallas import tpu_sc as plsc


def _copy_body(n_rows, chunk, n_sub, emb_hbm, out_hbm, buf, in_sem, out_sem):
    c = lax.axis_index("c")
    s = lax.axis_index("s")
    wid = s * 2 + c
    start = jnp.minimum(wid * chunk, n_rows - chunk)
    sub = chunk // n_sub
    ins = [
        pltpu.make_async_copy(
            emb_hbm.at[pl.ds(start + j * sub, sub)], buf.at[j], in_sem
        )
        for j in range(n_sub)
    ]
    outs = [
        pltpu.make_async_copy(
            buf.at[j], out_hbm.at[pl.ds(start + j * sub, sub)], out_sem
        )
        for j in range(n_sub)
    ]
    for cp in ins:
        cp.start()
    for j in range(n_sub):
        ins[j].wait()
        outs[j].start()
    for cp in outs:
        cp.wait()


def kernel(x, emb_table):
    n = x.shape[2]
    d = emb_table.shape[1]
    mesh = plsc.VectorSubcoreMesh(core_axis_name="c", subcore_axis_name="s")
    n_workers = 32
    chunk = (-(-n // n_workers) + 7) // 8 * 8
    n_sub = 4

    def body(emb_hbm, out_hbm, buf, in_sem, out_sem):
        _copy_body(n, chunk, n_sub, emb_hbm, out_hbm, buf, in_sem, out_sem)

    out = pl.kernel(
        body,
        out_type=jax.ShapeDtypeStruct((n, d), emb_table.dtype),
        mesh=mesh,
        scratch_types=[
            pltpu.VMEM((n_sub, chunk // n_sub, d), emb_table.dtype),
            pltpu.SemaphoreType.DMA,
            pltpu.SemaphoreType.DMA,
        ],
    )(emb_table)
    return out[None, None]


# speedup vs baseline: 21.5371x; 2.8268x over previous
import jax
import jax.numpy as jnp
from jax.experimental import pallas as pl


def _copy(in_ref, out_ref):
    out_ref[...] = in_ref[...]


def kernel(x, emb_table):
    n = x.shape[2]
    d = emb_table.shape[1]
    blk = 1000
    out = pl.pallas_call(
        _copy,
        grid=(n // blk,),
        in_specs=[pl.BlockSpec((blk, d), lambda i: (i, 0))],
        out_specs=pl.BlockSpec((blk, d), lambda i: (i, 0)),
        out_shape=jax.ShapeDtypeStruct((n, d), emb_table.dtype),
    )(emb_table)
    return out[None, None]
